# Initial kernel scaffold; baseline (speedup 1.0000x reference)
import jax
import jax.numpy as jnp
from jax.experimental import pallas as pl


def kernel(x, edge_index, edge_attr, node_type_emb, feat_W, feat_b, edge_emb0, W1_0, b1_0, W2_0, b2_0, edge_emb1, W1_1, b1_1, W2_1, b2_1):
    raise NotImplementedError("write your pallas kernel here")


# speedup vs baseline: 7.2956x; 7.2956x over previous
import jax
import jax.numpy as jnp
from jax import lax
from jax.experimental import pallas as pl
from jax.experimental.pallas import tpu as pltpu
from jax.experimental.pallas import tpu_sc as plsc

N = 10000
E = 320000
D = 128
HID = 2 * D
NT = 30
NET = 5

NC = 2
NS = 16
NW = NC * NS
EPW = E // NW
CHUNK = 80
NCHUNK = EPW // CHUNK
NP = 10240
RPT = NP // NS
CW = 16

_PREC = lax.Precision.HIGHEST


def _sc_body_agg(h_hbm, src_hbm, dst_hbm, znd_hbm,
                 parts_hbm,
                 src_v, dst_v, rows_v, acc_sh, sem):
    c = lax.axis_index("c")
    s = lax.axis_index("s")
    wid = s * NC + c
    r0 = s * RPT
    pltpu.sync_copy(znd_hbm.at[pl.ds(r0, RPT)], acc_sh.at[pl.ds(r0, RPT)])
    pltpu.sync_copy(src_hbm.at[wid], src_v)
    pltpu.sync_copy(dst_hbm.at[wid], dst_v)
    plsc.subcore_barrier()

    def step(j, carry):
        pltpu.async_copy(h_hbm.at[src_v.at[j]], rows_v, sem).wait()
        pltpu.sync_copy(rows_v, acc_sh.at[dst_v.at[j]], add=True)
        return carry

    lax.fori_loop(0, NCHUNK, step, 0)
    plsc.subcore_barrier()
    pltpu.sync_copy(acc_sh.at[pl.ds(r0, RPT)], parts_hbm.at[c].at[pl.ds(r0, RPT)])


def _sc_body_cnt(fidx_hbm, ones_hbm, zflat_hbm,
                 cflat_hbm,
                 fidx_v, ones_v, cnt_sh, sem):
    c = lax.axis_index("c")
    s = lax.axis_index("s")
    wid = s * NC + c
    f0 = s * RPT * CW
    pltpu.sync_copy(zflat_hbm.at[pl.ds(f0, RPT * CW)], cnt_sh.at[pl.ds(f0, RPT * CW)])
    pltpu.sync_copy(fidx_hbm.at[wid], fidx_v)
    pltpu.sync_copy(ones_hbm, ones_v)
    plsc.subcore_barrier()

    def step(j, carry):
        pltpu.sync_copy(ones_v, cnt_sh.at[fidx_v.at[j]], add=True)
        return carry

    lax.fori_loop(0, NCHUNK, step, 0)
    plsc.subcore_barrier()
    pltpu.sync_copy(cnt_sh.at[pl.ds(f0, RPT * CW)], cflat_hbm.at[c].at[pl.ds(f0, RPT * CW)])


_SC_MESH = plsc.VectorSubcoreMesh(core_axis_name="c", subcore_axis_name="s",
                                  num_cores=NC, num_subcores=NS)

_sc_agg = pl.kernel(
    _sc_body_agg,
    out_type=jax.ShapeDtypeStruct((NC, NP, D), jnp.float32),
    mesh=_SC_MESH,
    scratch_types=(
        pltpu.VMEM((NCHUNK, CHUNK), jnp.int32),
        pltpu.VMEM((NCHUNK, CHUNK), jnp.int32),
        pltpu.VMEM((CHUNK, D), jnp.float32),
        pltpu.VMEM_SHARED((NP, D), jnp.float32),
        pltpu.SemaphoreType.DMA,
    ),
    name="sc_edge_agg",
)

_sc_cnt = pl.kernel(
    _sc_body_cnt,
    out_type=jax.ShapeDtypeStruct((NC, NP * CW), jnp.float32),
    mesh=_SC_MESH,
    scratch_types=(
        pltpu.VMEM((NCHUNK, CHUNK), jnp.int32),
        pltpu.VMEM((CHUNK,), jnp.float32),
        pltpu.VMEM_SHARED((NP * CW,), jnp.float32),
        pltpu.SemaphoreType.DMA,
    ),
    name="sc_edge_cnt",
)


BN = 1000


def _embed_body(oh_ref, xf_ref, emb_ref, W_ref, b_ref, out_ref):
    out_ref[...] = (
        jnp.dot(oh_ref[...], emb_ref[...], preferred_element_type=jnp.float32,
                precision=_PREC)
        + jnp.dot(xf_ref[...], W_ref[...], preferred_element_type=jnp.float32,
                  precision=_PREC)
        + b_ref[...]
    )


_embed = pl.pallas_call(
    _embed_body,
    grid=(N // BN,),
    in_specs=[
        pl.BlockSpec((BN, 32), lambda i: (i, 0)),
        pl.BlockSpec((BN, D), lambda i: (i, 0)),
        pl.BlockSpec((32, D), lambda i: (0, 0)),
        pl.BlockSpec((D, D), lambda i: (0, 0)),
        pl.BlockSpec((1, D), lambda i: (0, 0)),
    ],
    out_specs=pl.BlockSpec((BN, D), lambda i: (i, 0)),
    out_shape=jax.ShapeDtypeStruct((N, D), jnp.float32),
)


def _mlp_body(relu_out, p0_ref, p1_ref, h_ref, c0_ref, c1_ref, ee_ref,
              W1_ref, b1_ref, W2_ref, b2_ref, out_ref):
    agg = (p0_ref[...] + p1_ref[...] + h_ref[...]
           + jnp.dot(c0_ref[...] + c1_ref[...], ee_ref[...],
                     preferred_element_type=jnp.float32, precision=_PREC))
    hid = jnp.maximum(
        jnp.dot(agg, W1_ref[...], preferred_element_type=jnp.float32,
                precision=_PREC) + b1_ref[...], 0.0)
    o = jnp.dot(hid, W2_ref[...], preferred_element_type=jnp.float32,
                precision=_PREC) + b2_ref[...]
    out_ref[...] = jnp.maximum(o, 0.0) if relu_out else o


def _make_mlp(relu_out):
    return pl.pallas_call(
        lambda *refs: _mlp_body(relu_out, *refs),
        grid=(N // BN,),
        in_specs=[
            pl.BlockSpec((BN, D), lambda i: (i, 0)),
            pl.BlockSpec((BN, D), lambda i: (i, 0)),
            pl.BlockSpec((BN, D), lambda i: (i, 0)),
            pl.BlockSpec((BN, CW), lambda i: (i, 0)),
            pl.BlockSpec((BN, CW), lambda i: (i, 0)),
            pl.BlockSpec((CW, D), lambda i: (0, 0)),
            pl.BlockSpec((D, HID), lambda i: (0, 0)),
            pl.BlockSpec((1, HID), lambda i: (0, 0)),
            pl.BlockSpec((HID, D), lambda i: (0, 0)),
            pl.BlockSpec((1, D), lambda i: (0, 0)),
        ],
        out_specs=pl.BlockSpec((BN, D), lambda i: (i, 0)),
        out_shape=jax.ShapeDtypeStruct((N, D), jnp.float32),
    )


_mlp_relu = _make_mlp(True)
_mlp_last = _make_mlp(False)


def kernel(x, edge_index, edge_attr, node_type_emb, feat_W, feat_b,
           edge_emb0, W1_0, b1_0, W2_0, b2_0,
           edge_emb1, W1_1, b1_1, W2_1, b2_1):
    ntype = jnp.clip(jnp.round(x[:, 0]), 0, NT - 1).astype(jnp.int32)
    oh_nt = (ntype[:, None] == jnp.arange(32, dtype=jnp.int32)[None, :]
             ).astype(jnp.float32)
    xf = x[:, 1:]
    emb32 = jnp.zeros((32, D), jnp.float32).at[:NT].set(node_type_emb)
    etype = jnp.clip(jnp.round(edge_attr[:, 0]), 0, NET - 1).astype(jnp.int32)
    src = edge_index[0].reshape(NW, NCHUNK, CHUNK)
    dst = edge_index[1].reshape(NW, NCHUNK, CHUNK)
    fidx = (edge_index[1] * CW + etype).reshape(NW, NCHUNK, CHUNK)
    ones_e = jnp.ones((CHUNK,), jnp.float32)
    znd = jnp.zeros((NP, D), jnp.float32)
    zflat = jnp.zeros((NP * CW,), jnp.float32)
    ee0 = jnp.zeros((CW, D), jnp.float32).at[:NET].set(edge_emb0)
    ee1 = jnp.zeros((CW, D), jnp.float32).at[:NET].set(edge_emb1)

    h0 = _embed(oh_nt, xf, emb32, feat_W, feat_b.reshape(1, D))
    cparts = _sc_cnt(fidx, ones_e, zflat).reshape(NC, NP, CW)
    parts0 = _sc_agg(h0, src, dst, znd)
    h1 = _mlp_relu(parts0[0], parts0[1], h0, cparts[0], cparts[1], ee0,
                   W1_0, b1_0.reshape(1, HID), W2_0, b2_0.reshape(1, D))
    parts1 = _sc_agg(h1, src, dst, znd)
    h2 = _mlp_last(parts1[0], parts1[1], h1, cparts[0], cparts[1], ee1,
                   W1_1, b1_1.reshape(1, HID), W2_1, b2_1.reshape(1, D))
    return h2


# speedup vs baseline: 8.1342x; 1.1149x over previous
import jax
import jax.numpy as jnp
from jax import lax
from jax.experimental import pallas as pl
from jax.experimental.pallas import tpu as pltpu
from jax.experimental.pallas import tpu_sc as plsc

N = 10000
E = 320000
D = 128
HID = 2 * D
NT = 30
NET = 5

NC = 2
NS = 16
NW = NC * NS
EPW = E // NW
CHUNK = 125
NCHUNK = EPW // CHUNK
NP = 10240
RPT = NP // NS
CW = 16

_PREC = lax.Precision.HIGHEST


def _sc_body_agg(h_hbm, src_hbm, dst_hbm, znd_hbm,
                 parts_hbm,
                 src_v, dst_v, rows_a, acc_sh, sem):
    c = lax.axis_index("c")
    s = lax.axis_index("s")
    wid = s * NC + c
    r0 = s * RPT
    pltpu.sync_copy(znd_hbm.at[pl.ds(r0, RPT)], acc_sh.at[pl.ds(r0, RPT)])
    pltpu.sync_copy(src_hbm.at[wid], src_v)
    pltpu.sync_copy(dst_hbm.at[wid], dst_v)
    plsc.subcore_barrier()

    def step(j, carry):
        pltpu.async_copy(h_hbm.at[src_v.at[j]], rows_a, sem).wait()
        pltpu.sync_copy(rows_a, acc_sh.at[dst_v.at[j]], add=True)
        return carry

    lax.fori_loop(0, NCHUNK, step, 0)

    plsc.subcore_barrier()
    pltpu.sync_copy(acc_sh.at[pl.ds(r0, RPT)], parts_hbm.at[c].at[pl.ds(r0, RPT)])


def _sc_body_cnt(fidx_hbm, ones_hbm, zflat_hbm,
                 cflat_hbm,
                 fidx_v, ones_v, cnt_sh, sem):
    c = lax.axis_index("c")
    s = lax.axis_index("s")
    wid = s * NC + c
    f0 = s * RPT * CW
    pltpu.sync_copy(zflat_hbm.at[pl.ds(f0, RPT * CW)], cnt_sh.at[pl.ds(f0, RPT * CW)])
    pltpu.sync_copy(fidx_hbm.at[wid], fidx_v)
    pltpu.sync_copy(ones_hbm, ones_v)
    plsc.subcore_barrier()

    def step(j, carry):
        pltpu.sync_copy(ones_v, cnt_sh.at[fidx_v.at[j]], add=True)
        return carry

    lax.fori_loop(0, NCHUNK, step, 0)
    plsc.subcore_barrier()
    pltpu.sync_copy(cnt_sh.at[pl.ds(f0, RPT * CW)], cflat_hbm.at[c].at[pl.ds(f0, RPT * CW)])


_SC_MESH = plsc.VectorSubcoreMesh(core_axis_name="c", subcore_axis_name="s",
                                  num_cores=NC, num_subcores=NS)

_sc_agg = pl.kernel(
    _sc_body_agg,
    out_type=jax.ShapeDtypeStruct((NC, NP, D), jnp.float32),
    mesh=_SC_MESH,
    scratch_types=(
        pltpu.VMEM((NCHUNK, CHUNK), jnp.int32),
        pltpu.VMEM((NCHUNK, CHUNK), jnp.int32),
        pltpu.VMEM((CHUNK, D), jnp.float32),
        pltpu.VMEM_SHARED((NP, D), jnp.float32),
        pltpu.SemaphoreType.DMA,
    ),
    name="sc_edge_agg",
)

_sc_cnt = pl.kernel(
    _sc_body_cnt,
    out_type=jax.ShapeDtypeStruct((NC, NP * CW), jnp.float32),
    mesh=_SC_MESH,
    scratch_types=(
        pltpu.VMEM((NCHUNK, CHUNK), jnp.int32),
        pltpu.VMEM((CHUNK,), jnp.float32),
        pltpu.VMEM_SHARED((NP * CW,), jnp.float32),
        pltpu.SemaphoreType.DMA,
    ),
    name="sc_edge_cnt",
)


BN = 1000


def _embed_body(oh_ref, xf_ref, emb_ref, W_ref, b_ref, out_ref):
    out_ref[...] = (
        jnp.dot(oh_ref[...], emb_ref[...], preferred_element_type=jnp.float32,
                precision=_PREC)
        + jnp.dot(xf_ref[...], W_ref[...], preferred_element_type=jnp.float32,
                  precision=_PREC)
        + b_ref[...]
    )


_embed = pl.pallas_call(
    _embed_body,
    grid=(N // BN,),
    in_specs=[
        pl.BlockSpec((BN, 32), lambda i: (i, 0)),
        pl.BlockSpec((BN, D), lambda i: (i, 0)),
        pl.BlockSpec((32, D), lambda i: (0, 0)),
        pl.BlockSpec((D, D), lambda i: (0, 0)),
        pl.BlockSpec((1, D), lambda i: (0, 0)),
    ],
    out_specs=pl.BlockSpec((BN, D), lambda i: (i, 0)),
    out_shape=jax.ShapeDtypeStruct((N, D), jnp.float32),
)


def _mlp_body(relu_out, p0_ref, p1_ref, h_ref, c0_ref, c1_ref, ee_ref,
              W1_ref, b1_ref, W2_ref, b2_ref, out_ref):
    agg = (p0_ref[...] + p1_ref[...] + h_ref[...]
           + jnp.dot(c0_ref[...] + c1_ref[...], ee_ref[...],
                     preferred_element_type=jnp.float32, precision=_PREC))
    hid = jnp.maximum(
        jnp.dot(agg, W1_ref[...], preferred_element_type=jnp.float32,
                precision=_PREC) + b1_ref[...], 0.0)
    o = jnp.dot(hid, W2_ref[...], preferred_element_type=jnp.float32,
                precision=_PREC) + b2_ref[...]
    out_ref[...] = jnp.maximum(o, 0.0) if relu_out else o


def _make_mlp(relu_out):
    return pl.pallas_call(
        lambda *refs: _mlp_body(relu_out, *refs),
        grid=(N // BN,),
        in_specs=[
            pl.BlockSpec((BN, D), lambda i: (i, 0)),
            pl.BlockSpec((BN, D), lambda i: (i, 0)),
            pl.BlockSpec((BN, D), lambda i: (i, 0)),
            pl.BlockSpec((BN, CW), lambda i: (i, 0)),
            pl.BlockSpec((BN, CW), lambda i: (i, 0)),
            pl.BlockSpec((CW, D), lambda i: (0, 0)),
            pl.BlockSpec((D, HID), lambda i: (0, 0)),
            pl.BlockSpec((1, HID), lambda i: (0, 0)),
            pl.BlockSpec((HID, D), lambda i: (0, 0)),
            pl.BlockSpec((1, D), lambda i: (0, 0)),
        ],
        out_specs=pl.BlockSpec((BN, D), lambda i: (i, 0)),
        out_shape=jax.ShapeDtypeStruct((N, D), jnp.float32),
    )


_mlp_relu = _make_mlp(True)
_mlp_last = _make_mlp(False)


def kernel(x, edge_index, edge_attr, node_type_emb, feat_W, feat_b,
           edge_emb0, W1_0, b1_0, W2_0, b2_0,
           edge_emb1, W1_1, b1_1, W2_1, b2_1):
    ntype = jnp.clip(jnp.round(x[:, 0]), 0, NT - 1).astype(jnp.int32)
    oh_nt = (ntype[:, None] == jnp.arange(32, dtype=jnp.int32)[None, :]
             ).astype(jnp.float32)
    xf = x[:, 1:]
    emb32 = jnp.zeros((32, D), jnp.float32).at[:NT].set(node_type_emb)
    etype = jnp.clip(jnp.round(edge_attr[:, 0]), 0, NET - 1).astype(jnp.int32)
    src = edge_index[0].reshape(NW, NCHUNK, CHUNK)
    dst = edge_index[1].reshape(NW, NCHUNK, CHUNK)
    fidx = (edge_index[1] * CW + etype).reshape(NW, NCHUNK, CHUNK)
    ones_e = jnp.ones((CHUNK,), jnp.float32)
    znd = jnp.zeros((NP, D), jnp.float32)
    zflat = jnp.zeros((NP * CW,), jnp.float32)
    ee0 = jnp.zeros((CW, D), jnp.float32).at[:NET].set(edge_emb0)
    ee1 = jnp.zeros((CW, D), jnp.float32).at[:NET].set(edge_emb1)

    h0 = _embed(oh_nt, xf, emb32, feat_W, feat_b.reshape(1, D))
    cparts = _sc_cnt(fidx, ones_e, zflat).reshape(NC, NP, CW)
    parts0 = _sc_agg(h0, src, dst, znd)
    h1 = _mlp_relu(parts0[0], parts0[1], h0, cparts[0], cparts[1], ee0,
                   W1_0, b1_0.reshape(1, HID), W2_0, b2_0.reshape(1, D))
    parts1 = _sc_agg(h1, src, dst, znd)
    h2 = _mlp_last(parts1[0], parts1[1], h1, cparts[0], cparts[1], ee1,
                   W1_1, b1_1.reshape(1, HID), W2_1, b2_1.reshape(1, D))
    return h2


# speedup vs baseline: 8.1578x; 1.0029x over previous
import jax
import jax.numpy as jnp
from jax import lax
from jax.experimental import pallas as pl
from jax.experimental.pallas import tpu as pltpu
from jax.experimental.pallas import tpu_sc as plsc

N = 10000
E = 320000
D = 128
HID = 2 * D
NT = 30
NET = 5

NC = 2
NS = 16
NW = NC * NS
EPW = E // NW
CHUNK = 125
NCHUNK = EPW // CHUNK
NP = 10240
RPT = NP // NS
CW = 5

_PREC = lax.Precision.HIGHEST


def _sc_body_agg(h_hbm, src_hbm, dst_hbm, znd_hbm,
                 parts_hbm,
                 src_v, dst_v, rows_a, acc_sh, sem):
    c = lax.axis_index("c")
    s = lax.axis_index("s")
    wid = s * NC + c
    r0 = s * RPT
    pltpu.sync_copy(znd_hbm.at[pl.ds(r0, RPT)], acc_sh.at[pl.ds(r0, RPT)])
    pltpu.sync_copy(src_hbm.at[wid], src_v)
    pltpu.sync_copy(dst_hbm.at[wid], dst_v)
    plsc.subcore_barrier()

    def step(j, carry):
        pltpu.async_copy(h_hbm.at[src_v.at[j]], rows_a, sem).wait()
        pltpu.sync_copy(rows_a, acc_sh.at[dst_v.at[j]], add=True)
        return carry

    lax.fori_loop(0, NCHUNK, step, 0)

    plsc.subcore_barrier()
    pltpu.sync_copy(acc_sh.at[pl.ds(r0, RPT)], parts_hbm.at[c].at[pl.ds(r0, RPT)])


def _sc_body_cnt(fidx_hbm, ones_hbm, zflat_hbm,
                 cflat_hbm,
                 fidx_v, ones_v, cnt_sh, sem):
    c = lax.axis_index("c")
    s = lax.axis_index("s")
    wid = s * NC + c
    f0 = s * RPT * CW
    pltpu.sync_copy(zflat_hbm.at[pl.ds(f0, RPT * CW)], cnt_sh.at[pl.ds(f0, RPT * CW)])
    pltpu.sync_copy(fidx_hbm.at[wid], fidx_v)
    pltpu.sync_copy(ones_hbm, ones_v)
    plsc.subcore_barrier()

    def step(j, carry):
        pltpu.sync_copy(ones_v, cnt_sh.at[fidx_v.at[j]], add=True)
        return carry

    lax.fori_loop(0, NCHUNK, step, 0)
    plsc.subcore_barrier()
    pltpu.sync_copy(cnt_sh.at[pl.ds(f0, RPT * CW)], cflat_hbm.at[c].at[pl.ds(f0, RPT * CW)])


_SC_MESH = plsc.VectorSubcoreMesh(core_axis_name="c", subcore_axis_name="s",
                                  num_cores=NC, num_subcores=NS)

_sc_agg = pl.kernel(
    _sc_body_agg,
    out_type=jax.ShapeDtypeStruct((NC, NP, D), jnp.float32),
    mesh=_SC_MESH,
    scratch_types=(
        pltpu.VMEM((NCHUNK, CHUNK), jnp.int32),
        pltpu.VMEM((NCHUNK, CHUNK), jnp.int32),
        pltpu.VMEM((CHUNK, D), jnp.float32),
        pltpu.VMEM_SHARED((NP, D), jnp.float32),
        pltpu.SemaphoreType.DMA,
    ),
    name="sc_edge_agg",
)

_sc_cnt = pl.kernel(
    _sc_body_cnt,
    out_type=jax.ShapeDtypeStruct((NC, NP * CW), jnp.float32),
    mesh=_SC_MESH,
    scratch_types=(
        pltpu.VMEM((NCHUNK, CHUNK), jnp.int32),
        pltpu.VMEM((CHUNK,), jnp.float32),
        pltpu.VMEM_SHARED((NP * CW,), jnp.float32),
        pltpu.SemaphoreType.DMA,
    ),
    name="sc_edge_cnt",
)


BN = 1000


def _embed_body(oh_ref, xf_ref, emb_ref, W_ref, b_ref, out_ref):
    out_ref[...] = (
        jnp.dot(oh_ref[...], emb_ref[...], preferred_element_type=jnp.float32,
                precision=_PREC)
        + jnp.dot(xf_ref[...], W_ref[...], preferred_element_type=jnp.float32,
                  precision=_PREC)
        + b_ref[...]
    )


_embed = pl.pallas_call(
    _embed_body,
    grid=(N // BN,),
    in_specs=[
        pl.BlockSpec((BN, 32), lambda i: (i, 0)),
        pl.BlockSpec((BN, D), lambda i: (i, 0)),
        pl.BlockSpec((32, D), lambda i: (0, 0)),
        pl.BlockSpec((D, D), lambda i: (0, 0)),
        pl.BlockSpec((1, D), lambda i: (0, 0)),
    ],
    out_specs=pl.BlockSpec((BN, D), lambda i: (i, 0)),
    out_shape=jax.ShapeDtypeStruct((N, D), jnp.float32),
)


def _mlp_body(relu_out, p0_ref, p1_ref, h_ref, c0_ref, c1_ref, ee_ref,
              W1_ref, b1_ref, W2_ref, b2_ref, out_ref):
    agg = (p0_ref[...] + p1_ref[...] + h_ref[...]
           + jnp.dot(c0_ref[...] + c1_ref[...], ee_ref[...],
                     preferred_element_type=jnp.float32, precision=_PREC))
    hid = jnp.maximum(
        jnp.dot(agg, W1_ref[...], preferred_element_type=jnp.float32,
                precision=_PREC) + b1_ref[...], 0.0)
    o = jnp.dot(hid, W2_ref[...], preferred_element_type=jnp.float32,
                precision=_PREC) + b2_ref[...]
    out_ref[...] = jnp.maximum(o, 0.0) if relu_out else o


def _make_mlp(relu_out):
    return pl.pallas_call(
        lambda *refs: _mlp_body(relu_out, *refs),
        grid=(N // BN,),
        in_specs=[
            pl.BlockSpec((BN, D), lambda i: (i, 0)),
            pl.BlockSpec((BN, D), lambda i: (i, 0)),
            pl.BlockSpec((BN, D), lambda i: (i, 0)),
            pl.BlockSpec((BN, CW), lambda i: (i, 0)),
            pl.BlockSpec((BN, CW), lambda i: (i, 0)),
            pl.BlockSpec((CW, D), lambda i: (0, 0)),
            pl.BlockSpec((D, HID), lambda i: (0, 0)),
            pl.BlockSpec((1, HID), lambda i: (0, 0)),
            pl.BlockSpec((HID, D), lambda i: (0, 0)),
            pl.BlockSpec((1, D), lambda i: (0, 0)),
        ],
        out_specs=pl.BlockSpec((BN, D), lambda i: (i, 0)),
        out_shape=jax.ShapeDtypeStruct((N, D), jnp.float32),
    )


_mlp_relu = _make_mlp(True)
_mlp_last = _make_mlp(False)


def kernel(x, edge_index, edge_attr, node_type_emb, feat_W, feat_b,
           edge_emb0, W1_0, b1_0, W2_0, b2_0,
           edge_emb1, W1_1, b1_1, W2_1, b2_1):
    ntype = jnp.clip(jnp.round(x[:, 0]), 0, NT - 1).astype(jnp.int32)
    oh_nt = (ntype[:, None] == jnp.arange(32, dtype=jnp.int32)[None, :]
             ).astype(jnp.float32)
    xf = x[:, 1:]
    emb32 = jnp.zeros((32, D), jnp.float32).at[:NT].set(node_type_emb)
    etype = jnp.clip(jnp.round(edge_attr[:, 0]), 0, NET - 1).astype(jnp.int32)
    src = edge_index[0].reshape(NW, NCHUNK, CHUNK)
    dst = edge_index[1].reshape(NW, NCHUNK, CHUNK)
    fidx = (edge_index[1] * CW + etype).reshape(NW, NCHUNK, CHUNK)
    ones_e = jnp.ones((CHUNK,), jnp.float32)
    znd = jnp.zeros((NP, D), jnp.float32)
    zflat = jnp.zeros((NP * CW,), jnp.float32)
    ee0 = jnp.zeros((CW, D), jnp.float32).at[:NET].set(edge_emb0)
    ee1 = jnp.zeros((CW, D), jnp.float32).at[:NET].set(edge_emb1)

    h0 = _embed(oh_nt, xf, emb32, feat_W, feat_b.reshape(1, D))
    cparts = _sc_cnt(fidx, ones_e, zflat).reshape(NC, NP, CW)
    parts0 = _sc_agg(h0, src, dst, znd)
    h1 = _mlp_relu(parts0[0], parts0[1], h0, cparts[0], cparts[1], ee0,
                   W1_0, b1_0.reshape(1, HID), W2_0, b2_0.reshape(1, D))
    parts1 = _sc_agg(h1, src, dst, znd)
    h2 = _mlp_last(parts1[0], parts1[1], h1, cparts[0], cparts[1], ee1,
                   W1_1, b1_1.reshape(1, HID), W2_1, b2_1.reshape(1, D))
    return h2


# speedup vs baseline: 9.2558x; 1.1346x over previous
import jax
import jax.numpy as jnp
from jax import lax
from jax.experimental import pallas as pl
from jax.experimental.pallas import tpu as pltpu
from jax.experimental.pallas import tpu_sc as plsc

N = 10000
E = 320000
D = 128
HID = 2 * D
NT = 30
NET = 5

NC = 2
NS = 16
NW = NC * NS
EPW = E // NW
CHUNK = 125
NCHUNK = EPW // CHUNK
NP = 10240
RPT = NP // NS
CW = 5

_PREC = lax.Precision.DEFAULT


def _sc_body_agg(h_hbm, src_hbm, dst_hbm, znd_hbm,
                 parts_hbm,
                 src_v, dst_v, rows_a, acc_sh, sem):
    c = lax.axis_index("c")
    s = lax.axis_index("s")
    wid = s * NC + c
    r0 = s * RPT
    pltpu.sync_copy(znd_hbm.at[pl.ds(r0, RPT)], acc_sh.at[pl.ds(r0, RPT)])
    pltpu.sync_copy(src_hbm.at[wid], src_v)
    pltpu.sync_copy(dst_hbm.at[wid], dst_v)
    plsc.subcore_barrier()

    def step(j, carry):
        pltpu.async_copy(h_hbm.at[src_v.at[j]], rows_a, sem).wait()
        pltpu.sync_copy(rows_a, acc_sh.at[dst_v.at[j]], add=True)
        return carry

    lax.fori_loop(0, NCHUNK, step, 0)

    plsc.subcore_barrier()
    pltpu.sync_copy(acc_sh.at[pl.ds(r0, RPT)], parts_hbm.at[c].at[pl.ds(r0, RPT)])


def _sc_body_cnt(fidx_hbm, ones_hbm, zflat_hbm,
                 cflat_hbm,
                 fidx_v, ones_v, cnt_sh, sem):
    c = lax.axis_index("c")
    s = lax.axis_index("s")
    wid = s * NC + c
    f0 = s * RPT * CW
    pltpu.sync_copy(zflat_hbm.at[pl.ds(f0, RPT * CW)], cnt_sh.at[pl.ds(f0, RPT * CW)])
    pltpu.sync_copy(fidx_hbm.at[wid], fidx_v)
    pltpu.sync_copy(ones_hbm, ones_v)
    plsc.subcore_barrier()

    def step(j, carry):
        pltpu.sync_copy(ones_v, cnt_sh.at[fidx_v.at[j]], add=True)
        return carry

    lax.fori_loop(0, NCHUNK, step, 0)
    plsc.subcore_barrier()
    pltpu.sync_copy(cnt_sh.at[pl.ds(f0, RPT * CW)], cflat_hbm.at[c].at[pl.ds(f0, RPT * CW)])


_SC_MESH = plsc.VectorSubcoreMesh(core_axis_name="c", subcore_axis_name="s",
                                  num_cores=NC, num_subcores=NS)

_sc_agg = pl.kernel(
    _sc_body_agg,
    out_type=jax.ShapeDtypeStruct((NC, NP, D), jnp.float32),
    mesh=_SC_MESH,
    scratch_types=(
        pltpu.VMEM((NCHUNK, CHUNK), jnp.int32),
        pltpu.VMEM((NCHUNK, CHUNK), jnp.int32),
        pltpu.VMEM((CHUNK, D), jnp.float32),
        pltpu.VMEM_SHARED((NP, D), jnp.float32),
        pltpu.SemaphoreType.DMA,
    ),
    name="sc_edge_agg",
)

_sc_cnt = pl.kernel(
    _sc_body_cnt,
    out_type=jax.ShapeDtypeStruct((NC, NP * CW), jnp.float32),
    mesh=_SC_MESH,
    scratch_types=(
        pltpu.VMEM((NCHUNK, CHUNK), jnp.int32),
        pltpu.VMEM((CHUNK,), jnp.float32),
        pltpu.VMEM_SHARED((NP * CW,), jnp.float32),
        pltpu.SemaphoreType.DMA,
    ),
    name="sc_edge_cnt",
)


BN = 1000


def _embed_body(oh_ref, xf_ref, emb_ref, W_ref, b_ref, out_ref):
    out_ref[...] = (
        jnp.dot(oh_ref[...], emb_ref[...], preferred_element_type=jnp.float32,
                precision=_PREC)
        + jnp.dot(xf_ref[...], W_ref[...], preferred_element_type=jnp.float32,
                  precision=_PREC)
        + b_ref[...]
    )


_embed = pl.pallas_call(
    _embed_body,
    grid=(N // BN,),
    in_specs=[
        pl.BlockSpec((BN, 32), lambda i: (i, 0)),
        pl.BlockSpec((BN, D), lambda i: (i, 0)),
        pl.BlockSpec((32, D), lambda i: (0, 0)),
        pl.BlockSpec((D, D), lambda i: (0, 0)),
        pl.BlockSpec((1, D), lambda i: (0, 0)),
    ],
    out_specs=pl.BlockSpec((BN, D), lambda i: (i, 0)),
    out_shape=jax.ShapeDtypeStruct((N, D), jnp.float32),
)


def _mlp_body(relu_out, p0_ref, p1_ref, h_ref, c0_ref, c1_ref, ee_ref,
              W1_ref, b1_ref, W2_ref, b2_ref, out_ref):
    agg = (p0_ref[...] + p1_ref[...] + h_ref[...]
           + jnp.dot(c0_ref[...] + c1_ref[...], ee_ref[...],
                     preferred_element_type=jnp.float32, precision=_PREC))
    hid = jnp.maximum(
        jnp.dot(agg, W1_ref[...], preferred_element_type=jnp.float32,
                precision=_PREC) + b1_ref[...], 0.0)
    o = jnp.dot(hid, W2_ref[...], preferred_element_type=jnp.float32,
                precision=_PREC) + b2_ref[...]
    out_ref[...] = jnp.maximum(o, 0.0) if relu_out else o


def _make_mlp(relu_out):
    return pl.pallas_call(
        lambda *refs: _mlp_body(relu_out, *refs),
        grid=(N // BN,),
        in_specs=[
            pl.BlockSpec((BN, D), lambda i: (i, 0)),
            pl.BlockSpec((BN, D), lambda i: (i, 0)),
            pl.BlockSpec((BN, D), lambda i: (i, 0)),
            pl.BlockSpec((BN, CW), lambda i: (i, 0)),
            pl.BlockSpec((BN, CW), lambda i: (i, 0)),
            pl.BlockSpec((CW, D), lambda i: (0, 0)),
            pl.BlockSpec((D, HID), lambda i: (0, 0)),
            pl.BlockSpec((1, HID), lambda i: (0, 0)),
            pl.BlockSpec((HID, D), lambda i: (0, 0)),
            pl.BlockSpec((1, D), lambda i: (0, 0)),
        ],
        out_specs=pl.BlockSpec((BN, D), lambda i: (i, 0)),
        out_shape=jax.ShapeDtypeStruct((N, D), jnp.float32),
    )


_mlp_relu = _make_mlp(True)
_mlp_last = _make_mlp(False)


def kernel(x, edge_index, edge_attr, node_type_emb, feat_W, feat_b,
           edge_emb0, W1_0, b1_0, W2_0, b2_0,
           edge_emb1, W1_1, b1_1, W2_1, b2_1):
    ntype = jnp.clip(jnp.round(x[:, 0]), 0, NT - 1).astype(jnp.int32)
    oh_nt = (ntype[:, None] == jnp.arange(32, dtype=jnp.int32)[None, :]
             ).astype(jnp.float32)
    xf = x[:, 1:]
    emb32 = jnp.zeros((32, D), jnp.float32).at[:NT].set(node_type_emb)
    etype = jnp.clip(jnp.round(edge_attr[:, 0]), 0, NET - 1).astype(jnp.int32)
    src = edge_index[0].reshape(NW, NCHUNK, CHUNK)
    dst = edge_index[1].reshape(NW, NCHUNK, CHUNK)
    fidx = (edge_index[1] * CW + etype).reshape(NW, NCHUNK, CHUNK)
    ones_e = jnp.ones((CHUNK,), jnp.float32)
    znd = jnp.zeros((NP, D), jnp.float32)
    zflat = jnp.zeros((NP * CW,), jnp.float32)
    ee0 = jnp.zeros((CW, D), jnp.float32).at[:NET].set(edge_emb0)
    ee1 = jnp.zeros((CW, D), jnp.float32).at[:NET].set(edge_emb1)

    h0 = _embed(oh_nt, xf, emb32, feat_W, feat_b.reshape(1, D))
    cparts = _sc_cnt(fidx, ones_e, zflat).reshape(NC, NP, CW)
    parts0 = _sc_agg(h0, src, dst, znd)
    h1 = _mlp_relu(parts0[0], parts0[1], h0, cparts[0], cparts[1], ee0,
                   W1_0, b1_0.reshape(1, HID), W2_0, b2_0.reshape(1, D))
    parts1 = _sc_agg(h1, src, dst, znd)
    h2 = _mlp_last(parts1[0], parts1[1], h1, cparts[0], cparts[1], ee1,
                   W1_1, b1_1.reshape(1, HID), W2_1, b2_1.reshape(1, D))
    return h2


# speedup vs baseline: 9.5622x; 1.0331x over previous
import jax
import jax.numpy as jnp
from jax import lax
from jax.experimental import pallas as pl
from jax.experimental.pallas import tpu as pltpu
from jax.experimental.pallas import tpu_sc as plsc

N = 10000
E = 320000
D = 128
HID = 2 * D
NT = 30
NET = 5

NC = 2
NS = 16
NW = NC * NS
EPW = E // NW
CHUNK = 125
NCHUNK = EPW // CHUNK
NP = 10240
RPT = NP // NS
CW = 5

_PREC = lax.Precision.DEFAULT


def _sc_body_agg(h_hbm, src_hbm, dst_hbm, znd_hbm,
                 parts_hbm,
                 src_v, dst_v, rows_a, acc_sh, sem):
    c = lax.axis_index("c")
    s = lax.axis_index("s")
    wid = s * NC + c
    r0 = s * RPT
    pltpu.sync_copy(znd_hbm.at[pl.ds(r0, RPT)], acc_sh.at[pl.ds(r0, RPT)])
    pltpu.sync_copy(src_hbm.at[wid], src_v)
    pltpu.sync_copy(dst_hbm.at[wid], dst_v)
    plsc.subcore_barrier()

    def step(j, carry):
        pltpu.async_copy(h_hbm.at[src_v.at[j]], rows_a, sem).wait()
        pltpu.sync_copy(rows_a, acc_sh.at[dst_v.at[j]], add=True)
        return carry

    lax.fori_loop(0, NCHUNK, step, 0)

    plsc.subcore_barrier()
    pltpu.sync_copy(acc_sh.at[pl.ds(r0, RPT)], parts_hbm.at[c].at[pl.ds(r0, RPT)])


def _sc_body_cnt(fidx_hbm, ones_hbm, zflat_hbm,
                 cflat_hbm,
                 fidx_v, ones_v, cnt_sh, sem):
    c = lax.axis_index("c")
    s = lax.axis_index("s")
    wid = s * NC + c
    f0 = s * RPT * CW
    pltpu.sync_copy(zflat_hbm.at[pl.ds(f0, RPT * CW)], cnt_sh.at[pl.ds(f0, RPT * CW)])
    pltpu.sync_copy(fidx_hbm.at[wid], fidx_v)
    pltpu.sync_copy(ones_hbm, ones_v)
    plsc.subcore_barrier()

    def step(j, carry):
        pltpu.sync_copy(ones_v, cnt_sh.at[fidx_v.at[j]], add=True)
        return carry

    lax.fori_loop(0, NCHUNK, step, 0)
    plsc.subcore_barrier()
    pltpu.sync_copy(cnt_sh.at[pl.ds(f0, RPT * CW)], cflat_hbm.at[c].at[pl.ds(f0, RPT * CW)])


_SC_MESH = plsc.VectorSubcoreMesh(core_axis_name="c", subcore_axis_name="s",
                                  num_cores=NC, num_subcores=NS)

_sc_agg = pl.kernel(
    _sc_body_agg,
    out_type=jax.ShapeDtypeStruct((NC, NP, D), jnp.float32),
    mesh=_SC_MESH,
    scratch_types=(
        pltpu.VMEM((NCHUNK, CHUNK), jnp.int32),
        pltpu.VMEM((NCHUNK, CHUNK), jnp.int32),
        pltpu.VMEM((CHUNK, D), jnp.float32),
        pltpu.VMEM_SHARED((NP, D), jnp.float32),
        pltpu.SemaphoreType.DMA,
    ),
    name="sc_edge_agg",
)

_sc_cnt = pl.kernel(
    _sc_body_cnt,
    out_type=jax.ShapeDtypeStruct((NC, NP * CW), jnp.float32),
    mesh=_SC_MESH,
    scratch_types=(
        pltpu.VMEM((NCHUNK, CHUNK), jnp.int32),
        pltpu.VMEM((CHUNK,), jnp.float32),
        pltpu.VMEM_SHARED((NP * CW,), jnp.float32),
        pltpu.SemaphoreType.DMA,
    ),
    name="sc_edge_cnt",
)


BN = 1000


def _embed_body(oh_ref, xf_ref, emb_ref, W_ref, b_ref, out_ref):
    out_ref[...] = (
        jnp.dot(oh_ref[...], emb_ref[...], preferred_element_type=jnp.float32,
                precision=_PREC)
        + jnp.dot(xf_ref[...], W_ref[...], preferred_element_type=jnp.float32,
                  precision=_PREC)
        + b_ref[...]
    )


_embed = pl.pallas_call(
    _embed_body,
    grid=(N // BN,),
    in_specs=[
        pl.BlockSpec((BN, 32), lambda i: (i, 0)),
        pl.BlockSpec((BN, D), lambda i: (i, 0)),
        pl.BlockSpec((32, D), lambda i: (0, 0)),
        pl.BlockSpec((D, D), lambda i: (0, 0)),
        pl.BlockSpec((1, D), lambda i: (0, 0)),
    ],
    out_specs=pl.BlockSpec((BN, D), lambda i: (i, 0)),
    out_shape=jax.ShapeDtypeStruct((N, D), jnp.float32),
)


def _mlp_body(relu_out, p0_ref, p1_ref, h_ref, c0_ref, c1_ref, ee_ref,
              W1_ref, b1_ref, W2_ref, b2_ref, out_ref):
    agg = (p0_ref[0] + p1_ref[0] + h_ref[...]
           + jnp.dot(c0_ref[0] + c1_ref[0], ee_ref[...],
                     preferred_element_type=jnp.float32, precision=_PREC))
    hid = jnp.maximum(
        jnp.dot(agg, W1_ref[...], preferred_element_type=jnp.float32,
                precision=_PREC) + b1_ref[...], 0.0)
    o = jnp.dot(hid, W2_ref[...], preferred_element_type=jnp.float32,
                precision=_PREC) + b2_ref[...]
    out_ref[...] = jnp.maximum(o, 0.0) if relu_out else o


def _make_mlp(relu_out):
    return pl.pallas_call(
        lambda *refs: _mlp_body(relu_out, *refs),
        grid=(N // BN,),
        in_specs=[
            pl.BlockSpec((1, BN, D), lambda i: (0, i, 0)),
            pl.BlockSpec((1, BN, D), lambda i: (1, i, 0)),
            pl.BlockSpec((BN, D), lambda i: (i, 0)),
            pl.BlockSpec((1, BN, CW), lambda i: (0, i, 0)),
            pl.BlockSpec((1, BN, CW), lambda i: (1, i, 0)),
            pl.BlockSpec((CW, D), lambda i: (0, 0)),
            pl.BlockSpec((D, HID), lambda i: (0, 0)),
            pl.BlockSpec((1, HID), lambda i: (0, 0)),
            pl.BlockSpec((HID, D), lambda i: (0, 0)),
            pl.BlockSpec((1, D), lambda i: (0, 0)),
        ],
        out_specs=pl.BlockSpec((BN, D), lambda i: (i, 0)),
        out_shape=jax.ShapeDtypeStruct((N, D), jnp.float32),
    )


_mlp_relu = _make_mlp(True)
_mlp_last = _make_mlp(False)


def kernel(x, edge_index, edge_attr, node_type_emb, feat_W, feat_b,
           edge_emb0, W1_0, b1_0, W2_0, b2_0,
           edge_emb1, W1_1, b1_1, W2_1, b2_1):
    ntype = jnp.clip(jnp.round(x[:, 0]), 0, NT - 1).astype(jnp.int32)
    oh_nt = (ntype[:, None] == jnp.arange(32, dtype=jnp.int32)[None, :]
             ).astype(jnp.float32)
    xf = x[:, 1:]
    emb32 = jnp.zeros((32, D), jnp.float32).at[:NT].set(node_type_emb)
    etype = jnp.clip(jnp.round(edge_attr[:, 0]), 0, NET - 1).astype(jnp.int32)
    src = edge_index[0].reshape(NW, NCHUNK, CHUNK)
    dst = edge_index[1].reshape(NW, NCHUNK, CHUNK)
    fidx = (edge_index[1] * CW + etype).reshape(NW, NCHUNK, CHUNK)
    ones_e = jnp.ones((CHUNK,), jnp.float32)
    znd = jnp.zeros((NP, D), jnp.float32)
    zflat = jnp.zeros((NP * CW,), jnp.float32)
    ee0 = jnp.zeros((CW, D), jnp.float32).at[:NET].set(edge_emb0)
    ee1 = jnp.zeros((CW, D), jnp.float32).at[:NET].set(edge_emb1)

    h0 = _embed(oh_nt, xf, emb32, feat_W, feat_b.reshape(1, D))
    cparts = _sc_cnt(fidx, ones_e, zflat).reshape(NC, NP, CW)
    parts0 = _sc_agg(h0, src, dst, znd)
    h1 = _mlp_relu(parts0, parts0, h0, cparts, cparts, ee0,
                   W1_0, b1_0.reshape(1, HID), W2_0, b2_0.reshape(1, D))
    parts1 = _sc_agg(h1, src, dst, znd)
    h2 = _mlp_last(parts1, parts1, h1, cparts, cparts, ee1,
                   W1_1, b1_1.reshape(1, HID), W2_1, b2_1.reshape(1, D))
    return h2
